# Initial kernel scaffold; baseline (speedup 1.0000x reference)
import jax
import jax.numpy as jnp
from jax.experimental import pallas as pl


def kernel(positions, x_table, y_table, z_table, W, b):
    raise NotImplementedError("write your pallas kernel here")


# speedup vs baseline: 1.5770x; 1.5770x over previous
import functools

import jax
import jax.numpy as jnp
from jax import lax
from jax.experimental import pallas as pl
from jax.experimental.pallas import tpu as pltpu
from jax.experimental.pallas import tpu_sc as plsc

VD = 128
DA = 32
D = 96
NJ = 6
NC = 2
NS = 16
NW = NC * NS
L = 16


def _build_fused_table(x_table, y_table, z_table, W, b2d):
    def body(xt, yt, zt, w, bb, out_ref):
        tabs = (xt, yt, zt)
        for j in range(NJ):
            blk = jnp.dot(
                tabs[j % 3][:],
                w[DA * j:DA * (j + 1), :],
                preferred_element_type=jnp.float32,
            )
            if j == 0:
                blk = blk + bb[:]
            out_ref[VD * j:VD * (j + 1), :] = blk

    return pl.pallas_call(
        body,
        out_shape=jax.ShapeDtypeStruct((NJ * VD, D), jnp.float32),
    )(x_table, y_table, z_table, W, b2d)


def _sc_gather_sum(pos_t, t_flat, batch):
    rows_per_w = batch // NW
    groups = rows_per_w // L
    mesh = plsc.VectorSubcoreMesh(
        core_axis_name="c", subcore_axis_name="s", num_cores=NC, num_subcores=NS
    )

    @functools.partial(
        pl.kernel,
        out_type=jax.ShapeDtypeStruct((batch * D,), jnp.float32),
        mesh=mesh,
        scratch_types=[
            pltpu.VMEM((NJ * VD * D,), jnp.float32),
            pltpu.VMEM((NJ, rows_per_w), jnp.int32),
            pltpu.VMEM((rows_per_w * D,), jnp.float32),
        ],
        compiler_params=pltpu.CompilerParams(needs_layout_passes=False),
    )
    def k(pos_hbm, t_hbm, out_hbm, t_v, pos_v, out_v):
        wid = lax.axis_index("s") * NC + lax.axis_index("c")
        base = wid * rows_per_w
        pltpu.sync_copy(t_hbm, t_v)
        pltpu.sync_copy(pos_hbm.at[:, pl.ds(base, rows_per_w)], pos_v)
        lanes = lax.iota(jnp.int32, L)

        @pl.loop(0, groups)
        def _(g):
            g16 = g * L
            bases = [
                pos_v[j, pl.ds(g16, L)] * D + (j * VD * D) for j in range(NJ)
            ]
            obase = (g16 + lanes) * D
            for c in range(D):
                acc = plsc.load_gather(t_v, [bases[0] + c])
                for j in range(1, NJ):
                    acc = acc + plsc.load_gather(t_v, [bases[j] + c])
                acc = jnp.maximum(acc, 0.0)
                plsc.store_scatter(out_v, [obase + c], acc)

        pltpu.sync_copy(out_v, out_hbm.at[pl.ds(base * D, rows_per_w * D)])

    return k(pos_t, t_flat)


def kernel(positions, x_table, y_table, z_table, W, b):
    batch = positions.shape[0]
    t = _build_fused_table(x_table, y_table, z_table, W, b.reshape(1, D))
    pos_t = positions.reshape(batch, NJ).T
    out_flat = _sc_gather_sum(pos_t, t.reshape(-1), batch)
    return out_flat.reshape(batch, D)


# speedup vs baseline: 2.5890x; 1.6418x over previous
import functools

import jax
import jax.numpy as jnp
from jax import lax
from jax.experimental import pallas as pl
from jax.experimental.pallas import tpu as pltpu
from jax.experimental.pallas import tpu_sc as plsc

VD = 128
DA = 32
D = 96
NJ = 6
NC = 2
NS = 16
NW = NC * NS
L = 16
IW = 128


def _build_fused_table(x_table, y_table, z_table, W, b2d):
    def body(xt, yt, zt, w, bb, out_ref):
        tabs = (xt, yt, zt)
        for j in range(NJ):
            blk = jnp.dot(
                tabs[j % 3][:],
                w[DA * j:DA * (j + 1), :],
                preferred_element_type=jnp.float32,
            )
            if j == 0:
                blk = blk + bb[:]
            out_ref[VD * j:VD * (j + 1), :] = blk

    return pl.pallas_call(
        body,
        out_shape=jax.ShapeDtypeStruct((NJ * VD, D), jnp.float32),
    )(x_table, y_table, z_table, W, b2d)


def _sc_gather_sum(pos2d, t, batch):
    rows_per_w = batch // NW
    ch = 128
    nch = rows_per_w // ch
    ir = ch * NJ // IW
    mesh = plsc.VectorSubcoreMesh(
        core_axis_name="c", subcore_axis_name="s", num_cores=NC, num_subcores=NS
    )

    wir = rows_per_w * NJ // IW

    @functools.partial(
        pl.kernel,
        out_type=jax.ShapeDtypeStruct((batch, D), jnp.float32),
        mesh=mesh,
        scratch_types=[
            pltpu.VMEM((wir, IW), jnp.int32),
            pltpu.VMEM((ch * NJ, D), jnp.float32),
            pltpu.VMEM((ch, D), jnp.float32),
            pltpu.SemaphoreType.DMA,
        ],
        compiler_params=pltpu.CompilerParams(
            needs_layout_passes=False, use_tc_tiling_on_sc=False
        ),
    )
    def k(pos_hbm, t_hbm, out_hbm, idx_v, gath_v, out_v, sem):
        wid = lax.axis_index("s") * NC + lax.axis_index("c")
        iota = lax.iota(jnp.int32, L)
        offs = {p: ((iota + p) % NJ) * VD for p in (0, 2, 4)}

        pltpu.sync_copy(pos_hbm.at[pl.ds(wid * wir, wir)], idx_v)
        for kk in range(wir):
            for s in range(IW // L):
                ph = (kk * IW + s * L) % NJ
                sl = idx_v[kk, pl.ds(s * L, L)]
                idx_v[kk, pl.ds(s * L, L)] = sl + offs[ph]

        @pl.loop(0, nch)
        def _(c):
            row0 = wid * rows_per_w + c * ch
            cps = [
                pltpu.async_copy(
                    t_hbm.at[idx_v.at[c * ir + kk]],
                    gath_v.at[pl.ds(kk * IW, IW), :],
                    sem,
                )
                for kk in range(ir)
            ]
            for cp in cps:
                cp.wait()

            @pl.loop(0, ch, unroll=4)
            def _(i):
                for cb in range(D // L):
                    acc = gath_v[i * NJ, pl.ds(cb * L, L)]
                    for j in range(1, NJ):
                        acc = acc + gath_v[i * NJ + j, pl.ds(cb * L, L)]
                    out_v[i, pl.ds(cb * L, L)] = jnp.maximum(acc, 0.0)

            pltpu.sync_copy(out_v, out_hbm.at[pl.ds(row0, ch), :])

    return k(pos2d, t)


def kernel(positions, x_table, y_table, z_table, W, b):
    batch = positions.shape[0]
    t = _build_fused_table(x_table, y_table, z_table, W, b.reshape(1, D))
    pos2d = positions.reshape(batch * NJ // IW, IW)
    return _sc_gather_sum(pos2d, t, batch)


# speedup vs baseline: 4.7263x; 1.8255x over previous
import functools

import jax
import jax.numpy as jnp
from jax import lax
from jax.experimental import pallas as pl
from jax.experimental.pallas import tpu as pltpu
from jax.experimental.pallas import tpu_sc as plsc

VD = 128
DA = 32
D = 96
NM = 6
NC = 2
NS = 16
NW = NC * NS
L = 16
IW = 128


def _build_fused_table(x_table, y_table, z_table, W, b2d):
    def body(xt, yt, zt, w, bb, out_ref):
        tabs = (xt, yt, zt)
        for m in range(NM):
            a, p = m // 2, m % 2
            j = p * 3 + a
            blk = jnp.dot(
                tabs[a][:],
                w[DA * j:DA * (j + 1), :],
                preferred_element_type=jnp.float32,
            )
            if m == 0:
                blk = blk + bb[:]
            out_ref[VD * m:VD * (m + 1), :] = blk

    return pl.pallas_call(
        body,
        out_shape=jax.ShapeDtypeStruct((NM * VD, D), jnp.float32),
    )(x_table, y_table, z_table, W, b2d)


def _sc_gather_sum(pos4d, t, batch):
    nblk = batch // IW
    bpw = nblk // NW
    mesh = plsc.VectorSubcoreMesh(
        core_axis_name="c", subcore_axis_name="s", num_cores=NC, num_subcores=NS
    )

    @functools.partial(
        pl.kernel,
        out_type=jax.ShapeDtypeStruct((batch, D), jnp.float32),
        mesh=mesh,
        scratch_types=[
            pltpu.VMEM((3, bpw, 2, IW), jnp.int32),
            pltpu.VMEM((NM * IW, D), jnp.float32),
            pltpu.VMEM((IW, D), jnp.float32),
            pltpu.SemaphoreType.DMA,
        ],
        compiler_params=pltpu.CompilerParams(
            needs_layout_passes=False, use_tc_tiling_on_sc=False
        ),
    )
    def k(pos_hbm, t_hbm, out_hbm, idx_v, gath_v, out_v, sem):
        wid = lax.axis_index("s") * NC + lax.axis_index("c")
        blk0 = wid * bpw
        pltpu.sync_copy(pos_hbm.at[:, pl.ds(blk0, bpw)], idx_v)
        for a in range(3):
            for p in range(2):
                off = (a * 2 + p) * VD
                for ib in range(bpw):
                    for s in range(IW // L):
                        sl = idx_v[a, ib, p, pl.ds(s * L, L)]
                        idx_v[a, ib, p, pl.ds(s * L, L)] = sl + off

        @pl.loop(0, bpw)
        def _(ib):
            cps = [
                pltpu.async_copy(
                    t_hbm.at[idx_v.at[a, ib, p]],
                    gath_v.at[pl.ds((a * 2 + p) * IW, IW), :],
                    sem,
                )
                for a in range(3)
                for p in range(2)
            ]
            for cp in cps:
                cp.wait()

            @pl.loop(0, IW, unroll=4)
            def _(e):
                for cb in range(D // L):
                    acc = gath_v[e, pl.ds(cb * L, L)]
                    for m in range(1, NM):
                        acc = acc + gath_v[m * IW + e, pl.ds(cb * L, L)]
                    out_v[e, pl.ds(cb * L, L)] = jnp.maximum(acc, 0.0)

            pltpu.sync_copy(out_v, out_hbm.at[pl.ds((blk0 + ib) * IW, IW), :])

    return k(pos4d, t)


def kernel(positions, x_table, y_table, z_table, W, b):
    batch = positions.shape[0]
    t = _build_fused_table(x_table, y_table, z_table, W, b.reshape(1, D))
    pos4d = (
        positions.transpose(2, 0, 1)
        .reshape(3, batch // IW, IW, 2)
        .transpose(0, 1, 3, 2)
    )
    return _sc_gather_sum(pos4d, t, batch)


# speedup vs baseline: 5.0301x; 1.0643x over previous
import functools

import jax
import jax.numpy as jnp
from jax import lax
from jax.experimental import pallas as pl
from jax.experimental.pallas import tpu as pltpu
from jax.experimental.pallas import tpu_sc as plsc

VD = 128
DA = 32
D = 96
NM = 6
NC = 2
NS = 16
NW = NC * NS
L = 16
IW = 128


def _build_fused_table(x_table, y_table, z_table, W, b2d):
    def body(xt, yt, zt, w, bb, out_ref):
        tabs = (xt, yt, zt)
        for m in range(NM):
            a, p = m // 2, m % 2
            j = p * 3 + a
            blk = jnp.dot(
                tabs[a][:],
                w[DA * j:DA * (j + 1), :],
                preferred_element_type=jnp.float32,
            )
            if m == 0:
                blk = blk + bb[:]
            out_ref[VD * m:VD * (m + 1), :] = blk

    return pl.pallas_call(
        body,
        out_shape=jax.ShapeDtypeStruct((NM * VD, D), jnp.float32),
    )(x_table, y_table, z_table, W, b2d)


def _sc_gather_sum(pos4d, t, batch):
    nblk = batch // IW
    bpw = nblk // NW
    mesh = plsc.VectorSubcoreMesh(
        core_axis_name="c", subcore_axis_name="s", num_cores=NC, num_subcores=NS
    )

    @functools.partial(
        pl.kernel,
        out_type=jax.ShapeDtypeStruct((batch, D), jnp.float32),
        mesh=mesh,
        scratch_types=[
            pltpu.VMEM((3, bpw, 2, IW), jnp.int32),
            pltpu.VMEM((2, 3 * IW, D), jnp.float32),
            pltpu.VMEM((2, IW, D), jnp.float32),
            pltpu.SemaphoreType.DMA,
            pltpu.SemaphoreType.DMA,
            pltpu.SemaphoreType.DMA,
            pltpu.SemaphoreType.DMA,
        ],
        compiler_params=pltpu.CompilerParams(
            needs_layout_passes=False, use_tc_tiling_on_sc=False
        ),
    )
    def k(pos_hbm, t_hbm, out_hbm, idx_v, gath_v, out_v, g0, g1, o0, o1, *_):
        wid = lax.axis_index("s") * NC + lax.axis_index("c")
        blk0 = wid * bpw
        gsem = (g0, g1)
        osem = (o0, o1)
        pltpu.sync_copy(pos_hbm.at[:, pl.ds(blk0, bpw)], idx_v)
        for a in range(3):
            for p in range(2):
                off = (a * 2 + p) * VD
                for ib in range(bpw):
                    for s in range(IW // L):
                        sl = idx_v[a, ib, p, pl.ds(s * L, L)]
                        idx_v[a, ib, p, pl.ds(s * L, L)] = sl + off

        def fire(g, buf):
            ib, p = g // 2, g % 2
            return [
                pltpu.async_copy(
                    t_hbm.at[idx_v.at[a, ib, p]],
                    gath_v.at[buf, pl.ds(a * IW, IW), :],
                    gsem[buf],
                )
                for a in range(3)
            ]

        ngroups = 2 * bpw
        odesc = [None, None]
        pend = fire(0, 0)
        for g in range(ngroups):
            ib, p = g // 2, g % 2
            buf = g % 2
            par = ib % 2
            nxt = fire(g + 1, 1 - buf) if g + 1 < ngroups else []
            for cp in pend:
                cp.wait()
            pend = nxt
            if p == 0 and odesc[par] is not None:
                odesc[par].wait()
                odesc[par] = None

            @pl.loop(0, IW, unroll=4)
            def _(e, p=p, buf=buf, par=par):
                for cb in range(D // L):
                    acc = gath_v[buf, e, pl.ds(cb * L, L)]
                    for a in range(1, 3):
                        acc = acc + gath_v[buf, a * IW + e, pl.ds(cb * L, L)]
                    if p == 0:
                        out_v[par, e, pl.ds(cb * L, L)] = acc
                    else:
                        prev = out_v[par, e, pl.ds(cb * L, L)]
                        out_v[par, e, pl.ds(cb * L, L)] = jnp.maximum(
                            prev + acc, 0.0
                        )

            if p == 1:
                odesc[par] = pltpu.async_copy(
                    out_v.at[par],
                    out_hbm.at[pl.ds((blk0 + ib) * IW, IW), :],
                    osem[par],
                )
        for dsc in odesc:
            if dsc is not None:
                dsc.wait()

    return k(pos4d, t)


def kernel(positions, x_table, y_table, z_table, W, b):
    batch = positions.shape[0]
    t = _build_fused_table(x_table, y_table, z_table, W, b.reshape(1, D))
    pos4d = (
        positions.transpose(2, 0, 1)
        .reshape(3, batch // IW, IW, 2)
        .transpose(0, 1, 3, 2)
    )
    return _sc_gather_sum(pos4d, t, batch)


# speedup vs baseline: 5.8541x; 1.1638x over previous
import functools

import jax
import jax.numpy as jnp
from jax import lax
from jax.experimental import pallas as pl
from jax.experimental.pallas import tpu as pltpu
from jax.experimental.pallas import tpu_sc as plsc

VD = 128
DA = 32
D = 96
NM = 6
NC = 2
NS = 16
NW = NC * NS
L = 16
IW = 128


def _build_fused_table(xt_t, yt_t, zt_t, W, b2d):
    def body(xt, yt, zt, w, bb, out_ref):
        tabs = (xt, yt, zt)
        for m in range(NM):
            a, p = m // 2, m % 2
            j = p * 3 + a
            blk = lax.dot_general(
                tabs[a][:],
                w[DA * j:DA * (j + 1), :],
                (((0,), (0,)), ((), ())),
                preferred_element_type=jnp.float32,
            )
            if m == 0:
                blk = blk + bb[:]
            out_ref[VD * m:VD * (m + 1), :] = blk

    return pl.pallas_call(
        body,
        out_shape=jax.ShapeDtypeStruct((NM * VD, D), jnp.float32),
    )(xt_t, yt_t, zt_t, W, b2d)


def _sc_gather_sum(pos4d, t, batch):
    nblk = batch // IW
    bpw = nblk // NW
    mesh = plsc.VectorSubcoreMesh(
        core_axis_name="c", subcore_axis_name="s", num_cores=NC, num_subcores=NS
    )

    @functools.partial(
        pl.kernel,
        out_type=jax.ShapeDtypeStruct((batch, IW), jnp.float32),
        mesh=mesh,
        scratch_types=[
            pltpu.VMEM((3, bpw, 2, IW), jnp.int32),
            pltpu.VMEM((2, 3 * IW, D), jnp.float32),
            pltpu.VMEM((2, IW, D), jnp.float32),
            pltpu.SemaphoreType.DMA,
            pltpu.SemaphoreType.DMA,
            pltpu.SemaphoreType.DMA,
            pltpu.SemaphoreType.DMA,
        ],
        compiler_params=pltpu.CompilerParams(
            needs_layout_passes=False, use_tc_tiling_on_sc=False
        ),
    )
    def k(pos_hbm, t_hbm, out_hbm, idx_v, gath_v, out_v, g0, g1, o0, o1, *_):
        wid = lax.axis_index("s") * NC + lax.axis_index("c")
        blk0 = wid * bpw
        gsem = (g0, g1)
        osem = (o0, o1)
        pltpu.sync_copy(pos_hbm.at[:, pl.ds(blk0, bpw)], idx_v)
        for a in range(3):
            for p in range(2):
                off = (a * 2 + p) * VD
                for ib in range(bpw):
                    for s in range(IW // L):
                        sl = idx_v[a, ib, p, pl.ds(s * L, L)]
                        idx_v[a, ib, p, pl.ds(s * L, L)] = sl + off

        def fire(g, buf):
            ib, p = g // 2, g % 2
            return [
                pltpu.async_copy(
                    t_hbm.at[idx_v.at[a, ib, p]],
                    gath_v.at[buf, pl.ds(a * IW, IW), :],
                    gsem[buf],
                )
                for a in range(3)
            ]

        ngroups = 2 * bpw
        odesc = [None, None]
        pend = fire(0, 0)
        for g in range(ngroups):
            ib, p = g // 2, g % 2
            buf = g % 2
            par = ib % 2
            nxt = fire(g + 1, 1 - buf) if g + 1 < ngroups else []
            for cp in pend:
                cp.wait()
            pend = nxt
            if p == 0 and odesc[par] is not None:
                odesc[par].wait()
                odesc[par] = None

            @pl.loop(0, IW, unroll=4)
            def _(e, p=p, buf=buf, par=par):
                for cb in range(D // L):
                    acc = gath_v[buf, e, pl.ds(cb * L, L)]
                    for a in range(1, 3):
                        acc = acc + gath_v[buf, a * IW + e, pl.ds(cb * L, L)]
                    if p == 0:
                        out_v[par, e, pl.ds(cb * L, L)] = acc
                    else:
                        prev = out_v[par, e, pl.ds(cb * L, L)]
                        out_v[par, e, pl.ds(cb * L, L)] = jnp.maximum(
                            prev + acc, 0.0
                        )

            if p == 1:
                odesc[par] = pltpu.async_copy(
                    out_v.at[par],
                    out_hbm.at[pl.ds((blk0 + ib) * IW, IW), pl.ds(0, D)],
                    osem[par],
                )
        for dsc in odesc:
            if dsc is not None:
                dsc.wait()

    return k(pos4d, t)


def kernel(positions, x_table, y_table, z_table, W, b):
    batch = positions.shape[0]
    t = _build_fused_table(
        x_table.T, y_table.T, z_table.T, W, b.reshape(1, D)
    )
    pos4d = (
        positions.transpose(2, 0, 1)
        .reshape(3, batch // IW, IW, 2)
        .transpose(0, 1, 3, 2)
    )
    out128 = _sc_gather_sum(pos4d, t, batch)
    return out128[:, :D]
